# Initial kernel scaffold; baseline (speedup 1.0000x reference)
import jax
import jax.numpy as jnp
from jax.experimental import pallas as pl


def kernel(xyz, points, W0, b0, g0, be0, W1, b1, g1, be1, W2, b2, g2, be2):
    raise NotImplementedError("write your pallas kernel here")


# speedup vs baseline: 1.0004x; 1.0004x over previous
import functools

import jax
import jax.numpy as jnp
import numpy as np
from jax.experimental import pallas as pl
from jax.experimental.pallas import tpu as pltpu

_NPOINT = 1024
_RADIUS = 0.4
_NSAMPLE = 32
_EPS = 1e-5


def _fps_jnp(xyz_t, npoint):
    B, N, _ = xyz_t.shape
    def body(i, state):
        centroids, distance, farthest = state
        centroids = centroids.at[:, i].set(farthest)
        centroid = xyz_t[jnp.arange(B), farthest, :][:, None, :]
        dist = jnp.sum((xyz_t - centroid) ** 2, axis=-1)
        distance = jnp.minimum(distance, dist)
        farthest = jnp.argmax(distance, axis=-1).astype(jnp.int32)
        return (centroids, distance, farthest)
    centroids = jnp.zeros((B, npoint), dtype=jnp.int32)
    distance = jnp.full((B, N), 1e10, dtype=jnp.float32)
    farthest = jnp.zeros((B,), dtype=jnp.int32)
    centroids, _, _ = jax.lax.fori_loop(0, npoint, body, (centroids, distance, farthest))
    return centroids


def _ball_query_jnp(radius, nsample, xyz_t, new_xyz):
    B, N, _ = xyz_t.shape
    S = new_xyz.shape[1]
    group_idx = jnp.broadcast_to(jnp.arange(N, dtype=jnp.int32)[None, None, :], (B, S, N))
    d = (jnp.sum(new_xyz ** 2, -1)[:, :, None] + jnp.sum(xyz_t ** 2, -1)[:, None, :]
         - 2.0 * jnp.einsum('bsc,bnc->bsn', new_xyz, xyz_t))
    d = jnp.maximum(d, 0.0)
    group_idx = jnp.where(d > radius ** 2, N, group_idx)
    group_idx = jnp.sort(group_idx, axis=-1)[:, :, :nsample]
    group_first = jnp.broadcast_to(group_idx[:, :, 0:1], (B, S, nsample))
    group_idx = jnp.where(group_idx == N, group_first, group_idx)
    return group_idx


_TR = 2048


def _layer0_body(f_ref, x_ref, wf_ref, wx_ref, z_ref, ps_ref):
    z = jnp.dot(x_ref[...], wx_ref[...], preferred_element_type=jnp.float32)
    z = z + jnp.dot(f_ref[...], wf_ref[...], preferred_element_type=jnp.float32)
    z_ref[...] = z
    ps_ref[...] = jnp.stack([jnp.sum(z, axis=0), jnp.sum(z * z, axis=0)])[None]


def _layer0(F, X4, W0f_t, W0x_t):
    R = F.shape[0]
    grid = (R // _TR,)
    return pl.pallas_call(
        _layer0_body,
        grid=grid,
        in_specs=[
            pl.BlockSpec((_TR, 64), lambda i: (i, 0)),
            pl.BlockSpec((_TR, 8), lambda i: (i, 0)),
            pl.BlockSpec((64, 64), lambda i: (0, 0)),
            pl.BlockSpec((8, 64), lambda i: (0, 0)),
        ],
        out_specs=[
            pl.BlockSpec((_TR, 64), lambda i: (i, 0)),
            pl.BlockSpec((1, 2, 64), lambda i: (i, 0, 0)),
        ],
        out_shape=[
            jax.ShapeDtypeStruct((R, 64), jnp.float32),
            jax.ShapeDtypeStruct((R // _TR, 2, 64), jnp.float32),
        ],
    )(F, X4, W0f_t, W0x_t)


def _layer_mid_body(z_ref, p_ref, w_ref, z1_ref, ps_ref):
    x = jax.nn.relu(z_ref[...] * p_ref[0][None, :] + p_ref[1][None, :])
    z1 = jnp.dot(x, w_ref[...], preferred_element_type=jnp.float32)
    z1_ref[...] = z1
    ps_ref[...] = jnp.stack([jnp.sum(z1, axis=0), jnp.sum(z1 * z1, axis=0)])[None]


def _layer_mid(Z0, params0, W1_t):
    R = Z0.shape[0]
    grid = (R // _TR,)
    return pl.pallas_call(
        _layer_mid_body,
        grid=grid,
        in_specs=[
            pl.BlockSpec((_TR, 64), lambda i: (i, 0)),
            pl.BlockSpec((8, 64), lambda i: (0, 0)),
            pl.BlockSpec((64, 64), lambda i: (0, 0)),
        ],
        out_specs=[
            pl.BlockSpec((_TR, 64), lambda i: (i, 0)),
            pl.BlockSpec((1, 2, 64), lambda i: (i, 0, 0)),
        ],
        out_shape=[
            jax.ShapeDtypeStruct((R, 64), jnp.float32),
            jax.ShapeDtypeStruct((R // _TR, 2, 64), jnp.float32),
        ],
    )(Z0, params0, W1_t)


def _layer_last_body(z_ref, p_ref, w_ref, m_ref, ps_ref):
    x = jax.nn.relu(z_ref[...] * p_ref[0][None, :] + p_ref[1][None, :])
    z2 = jnp.dot(x, w_ref[...], preferred_element_type=jnp.float32)
    ps_ref[...] = jnp.stack([jnp.sum(z2, axis=0), jnp.sum(z2 * z2, axis=0)])[None]
    m_ref[...] = jnp.max(z2.reshape(_TR // _NSAMPLE, _NSAMPLE, 128), axis=1)


def _layer_last(Z1, params1, W2_t):
    R = Z1.shape[0]
    grid = (R // _TR,)
    G = _TR // _NSAMPLE
    return pl.pallas_call(
        _layer_last_body,
        grid=grid,
        in_specs=[
            pl.BlockSpec((_TR, 64), lambda i: (i, 0)),
            pl.BlockSpec((8, 64), lambda i: (0, 0)),
            pl.BlockSpec((64, 128), lambda i: (0, 0)),
        ],
        out_specs=[
            pl.BlockSpec((G, 128), lambda i: (i, 0)),
            pl.BlockSpec((1, 2, 128), lambda i: (i, 0, 0)),
        ],
        out_shape=[
            jax.ShapeDtypeStruct((R // _NSAMPLE, 128), jnp.float32),
            jax.ShapeDtypeStruct((R // _TR, 2, 128), jnp.float32),
        ],
    )(Z1, params1, W2_t)


_TS_OUT = 256


def _finalize_body(m_ref, p_ref, o_ref):
    y = jax.nn.relu(m_ref[...] * p_ref[0][None, :] + p_ref[1][None, :])
    o_ref[...] = y.T[None]


def _finalize(M, params2, B, S):
    grid = (B, S // _TS_OUT)
    return pl.pallas_call(
        _finalize_body,
        grid=grid,
        in_specs=[
            pl.BlockSpec((_TS_OUT, 128), lambda b, j: (b * (S // _TS_OUT) + j, 0)),
            pl.BlockSpec((8, 128), lambda b, j: (0, 0)),
        ],
        out_specs=pl.BlockSpec((1, 128, _TS_OUT), lambda b, j: (b, 0, j)),
        out_shape=jax.ShapeDtypeStruct((B, 128, S), jnp.float32),
    )(M, params2)


def _bn_params(psum, count, g, be, cout):
    s = jnp.sum(psum, axis=0)
    mean = s[0] / count
    var = s[1] / count - mean * mean
    inv = g * jax.lax.rsqrt(var + _EPS)
    a = inv
    c = be - mean * inv
    pad = jnp.zeros((6, cout), jnp.float32)
    return jnp.concatenate([a[None], c[None], pad], axis=0)


def kernel(xyz, points, W0, b0, g0, be0, W1, b1, g1, be1, W2, b2, g2, be2):
    B, _, N = xyz.shape
    C = points.shape[1]
    S, K = _NPOINT, _NSAMPLE

    xyz_t = jnp.transpose(xyz, (0, 2, 1))
    pts_t = jnp.transpose(points, (0, 2, 1))

    fps_idx = _fps_jnp(xyz_t, S)
    new_xyz = jax.vmap(lambda p, i: p[i])(xyz_t, fps_idx)
    idx = _ball_query_jnp(_RADIUS, K, xyz_t, new_xyz)

    grouped_xyz = jax.vmap(lambda p, i: p[i])(xyz_t, idx.reshape(B, S * K))
    grouped_xyz = grouped_xyz.reshape(B, S, K, 3) - new_xyz[:, :, None, :]
    grouped_pts = jax.vmap(lambda p, i: p[i])(pts_t, idx.reshape(B, S * K))

    R = B * S * K
    F = grouped_pts.reshape(R, C)
    X4 = jnp.concatenate(
        [grouped_xyz.reshape(R, 3), jnp.zeros((R, 5), jnp.float32)], axis=1)

    W0x_t = jnp.concatenate([W0[:, :3].T, jnp.zeros((5, 64), jnp.float32)], axis=0)
    W0f_t = W0[:, 3:].T

    Z0, ps0 = _layer0(F, X4, W0f_t, W0x_t)
    p0 = _bn_params(ps0, R, g0, be0, 64)
    Z1, ps1 = _layer_mid(Z0, p0, W1.T)
    p1 = _bn_params(ps1, R, g1, be1, 64)
    M, ps2 = _layer_last(Z1, p1, W2.T)
    p2 = _bn_params(ps2, R, g2, be2, 128)
    new_points = _finalize(M, p2, B, S)

    return (jnp.transpose(new_xyz, (0, 2, 1)), new_points)


# speedup vs baseline: 1.5471x; 1.5464x over previous
import functools

import jax
import jax.numpy as jnp
import numpy as np
from jax.experimental import pallas as pl
from jax.experimental.pallas import tpu as pltpu

_NPOINT = 1024
_RADIUS = 0.4
_NSAMPLE = 32
_EPS = 1e-5


def _fps_body(xyz_ref, idx_ref, nxx_ref, nxy_ref, nxz_ref):
    B = xyz_ref.shape[0]
    N = xyz_ref.shape[2]
    S = _NPOINT
    x = xyz_ref[:, 0, :]
    y = xyz_ref[:, 1, :]
    z = xyz_ref[:, 2, :]
    iota = jax.lax.broadcasted_iota(jnp.int32, (B, N), 1)
    iota_s = jax.lax.broadcasted_iota(jnp.int32, (B, S), 1)

    def step(i, carry):
        dist, far, cen, ax, ay, az = carry
        slot = iota_s == i
        cen = jnp.where(slot, far, cen)
        onehot = iota == far
        cx = jnp.sum(jnp.where(onehot, x, 0.0), axis=1, keepdims=True)
        cy = jnp.sum(jnp.where(onehot, y, 0.0), axis=1, keepdims=True)
        cz = jnp.sum(jnp.where(onehot, z, 0.0), axis=1, keepdims=True)
        ax = jnp.where(slot, cx, ax)
        ay = jnp.where(slot, cy, ay)
        az = jnp.where(slot, cz, az)
        dx = x - cx
        dy = y - cy
        dz = z - cz
        d = dx * dx + dy * dy + dz * dz
        dist = jnp.minimum(dist, d)
        m = jnp.max(dist, axis=1, keepdims=True)
        far = jnp.min(jnp.where(dist == m, iota, N), axis=1, keepdims=True)
        return dist, far, cen, ax, ay, az

    dist0 = jnp.full((B, N), 1e10, jnp.float32) + x * 0.0
    far0 = (x[:, :1] * 0.0).astype(jnp.int32)
    zs = x[:, :S] * 0.0
    cen0 = zs.astype(jnp.int32)
    _, _, cen, ax, ay, az = jax.lax.fori_loop(
        0, S, step, (dist0, far0, cen0, zs, zs, zs))
    idx_ref[...] = cen
    nxx_ref[...] = ax
    nxy_ref[...] = ay
    nxz_ref[...] = az


def _fps_pallas(xyz):
    B, _, N = xyz.shape
    S = _NPOINT
    return pl.pallas_call(
        _fps_body,
        grid=(1,),
        in_specs=[pl.BlockSpec((B, 3, N), lambda i: (0, 0, 0))],
        out_specs=[
            pl.BlockSpec((B, S), lambda i: (0, 0)),
            pl.BlockSpec((B, S), lambda i: (0, 0)),
            pl.BlockSpec((B, S), lambda i: (0, 0)),
            pl.BlockSpec((B, S), lambda i: (0, 0)),
        ],
        out_shape=[
            jax.ShapeDtypeStruct((B, S), jnp.int32),
            jax.ShapeDtypeStruct((B, S), jnp.float32),
            jax.ShapeDtypeStruct((B, S), jnp.float32),
            jax.ShapeDtypeStruct((B, S), jnp.float32),
        ],
    )(xyz)


def _ball_query_jnp(radius, nsample, xyz_t, new_xyz):
    B, N, _ = xyz_t.shape
    S = new_xyz.shape[1]
    group_idx = jnp.broadcast_to(jnp.arange(N, dtype=jnp.int32)[None, None, :], (B, S, N))
    d = (jnp.sum(new_xyz ** 2, -1)[:, :, None] + jnp.sum(xyz_t ** 2, -1)[:, None, :]
         - 2.0 * jnp.einsum('bsc,bnc->bsn', new_xyz, xyz_t))
    d = jnp.maximum(d, 0.0)
    group_idx = jnp.where(d > radius ** 2, N, group_idx)
    group_idx = jnp.sort(group_idx, axis=-1)[:, :, :nsample]
    group_first = jnp.broadcast_to(group_idx[:, :, 0:1], (B, S, nsample))
    group_idx = jnp.where(group_idx == N, group_first, group_idx)
    return group_idx


_TR = 2048


def _layer0_body(f_ref, x_ref, wf_ref, wx_ref, z_ref, ps_ref):
    z = jnp.dot(x_ref[...], wx_ref[...], preferred_element_type=jnp.float32)
    z = z + jnp.dot(f_ref[...], wf_ref[...], preferred_element_type=jnp.float32)
    z_ref[...] = z
    ps_ref[...] = jnp.stack([jnp.sum(z, axis=0), jnp.sum(z * z, axis=0)])[None]


def _layer0(F, X4, W0f_t, W0x_t):
    R = F.shape[0]
    grid = (R // _TR,)
    return pl.pallas_call(
        _layer0_body,
        grid=grid,
        in_specs=[
            pl.BlockSpec((_TR, 64), lambda i: (i, 0)),
            pl.BlockSpec((_TR, 8), lambda i: (i, 0)),
            pl.BlockSpec((64, 64), lambda i: (0, 0)),
            pl.BlockSpec((8, 64), lambda i: (0, 0)),
        ],
        out_specs=[
            pl.BlockSpec((_TR, 64), lambda i: (i, 0)),
            pl.BlockSpec((1, 2, 64), lambda i: (i, 0, 0)),
        ],
        out_shape=[
            jax.ShapeDtypeStruct((R, 64), jnp.float32),
            jax.ShapeDtypeStruct((R // _TR, 2, 64), jnp.float32),
        ],
    )(F, X4, W0f_t, W0x_t)


def _layer_mid_body(z_ref, p_ref, w_ref, z1_ref, ps_ref):
    x = jax.nn.relu(z_ref[...] * p_ref[0][None, :] + p_ref[1][None, :])
    z1 = jnp.dot(x, w_ref[...], preferred_element_type=jnp.float32)
    z1_ref[...] = z1
    ps_ref[...] = jnp.stack([jnp.sum(z1, axis=0), jnp.sum(z1 * z1, axis=0)])[None]


def _layer_mid(Z0, params0, W1_t):
    R = Z0.shape[0]
    grid = (R // _TR,)
    return pl.pallas_call(
        _layer_mid_body,
        grid=grid,
        in_specs=[
            pl.BlockSpec((_TR, 64), lambda i: (i, 0)),
            pl.BlockSpec((8, 64), lambda i: (0, 0)),
            pl.BlockSpec((64, 64), lambda i: (0, 0)),
        ],
        out_specs=[
            pl.BlockSpec((_TR, 64), lambda i: (i, 0)),
            pl.BlockSpec((1, 2, 64), lambda i: (i, 0, 0)),
        ],
        out_shape=[
            jax.ShapeDtypeStruct((R, 64), jnp.float32),
            jax.ShapeDtypeStruct((R // _TR, 2, 64), jnp.float32),
        ],
    )(Z0, params0, W1_t)


def _layer_last_body(z_ref, p_ref, w_ref, m_ref, ps_ref):
    x = jax.nn.relu(z_ref[...] * p_ref[0][None, :] + p_ref[1][None, :])
    z2 = jnp.dot(x, w_ref[...], preferred_element_type=jnp.float32)
    ps_ref[...] = jnp.stack([jnp.sum(z2, axis=0), jnp.sum(z2 * z2, axis=0)])[None]
    m_ref[...] = jnp.max(z2.reshape(_TR // _NSAMPLE, _NSAMPLE, 128), axis=1)


def _layer_last(Z1, params1, W2_t):
    R = Z1.shape[0]
    grid = (R // _TR,)
    G = _TR // _NSAMPLE
    return pl.pallas_call(
        _layer_last_body,
        grid=grid,
        in_specs=[
            pl.BlockSpec((_TR, 64), lambda i: (i, 0)),
            pl.BlockSpec((8, 64), lambda i: (0, 0)),
            pl.BlockSpec((64, 128), lambda i: (0, 0)),
        ],
        out_specs=[
            pl.BlockSpec((G, 128), lambda i: (i, 0)),
            pl.BlockSpec((1, 2, 128), lambda i: (i, 0, 0)),
        ],
        out_shape=[
            jax.ShapeDtypeStruct((R // _NSAMPLE, 128), jnp.float32),
            jax.ShapeDtypeStruct((R // _TR, 2, 128), jnp.float32),
        ],
    )(Z1, params1, W2_t)


_TS_OUT = 256


def _finalize_body(m_ref, p_ref, o_ref):
    y = jax.nn.relu(m_ref[...] * p_ref[0][None, :] + p_ref[1][None, :])
    o_ref[...] = y.T[None]


def _finalize(M, params2, B, S):
    grid = (B, S // _TS_OUT)
    return pl.pallas_call(
        _finalize_body,
        grid=grid,
        in_specs=[
            pl.BlockSpec((_TS_OUT, 128), lambda b, j: (b * (S // _TS_OUT) + j, 0)),
            pl.BlockSpec((8, 128), lambda b, j: (0, 0)),
        ],
        out_specs=pl.BlockSpec((1, 128, _TS_OUT), lambda b, j: (b, 0, j)),
        out_shape=jax.ShapeDtypeStruct((B, 128, S), jnp.float32),
    )(M, params2)


def _bn_params(psum, count, g, be, cout):
    s = jnp.sum(psum, axis=0)
    mean = s[0] / count
    var = s[1] / count - mean * mean
    inv = g * jax.lax.rsqrt(var + _EPS)
    a = inv
    c = be - mean * inv
    pad = jnp.zeros((6, cout), jnp.float32)
    return jnp.concatenate([a[None], c[None], pad], axis=0)


def kernel(xyz, points, W0, b0, g0, be0, W1, b1, g1, be1, W2, b2, g2, be2):
    B, _, N = xyz.shape
    C = points.shape[1]
    S, K = _NPOINT, _NSAMPLE

    xyz_t = jnp.transpose(xyz, (0, 2, 1))
    pts_t = jnp.transpose(points, (0, 2, 1))

    fps_idx, nxx, nxy, nxz = _fps_pallas(xyz)
    new_xyz = jnp.stack([nxx, nxy, nxz], axis=2)
    idx = _ball_query_jnp(_RADIUS, K, xyz_t, new_xyz)

    grouped_xyz = jax.vmap(lambda p, i: p[i])(xyz_t, idx.reshape(B, S * K))
    grouped_xyz = grouped_xyz.reshape(B, S, K, 3) - new_xyz[:, :, None, :]
    grouped_pts = jax.vmap(lambda p, i: p[i])(pts_t, idx.reshape(B, S * K))

    R = B * S * K
    F = grouped_pts.reshape(R, C)
    X4 = jnp.concatenate(
        [grouped_xyz.reshape(R, 3), jnp.zeros((R, 5), jnp.float32)], axis=1)

    W0x_t = jnp.concatenate([W0[:, :3].T, jnp.zeros((5, 64), jnp.float32)], axis=0)
    W0f_t = W0[:, 3:].T

    Z0, ps0 = _layer0(F, X4, W0f_t, W0x_t)
    p0 = _bn_params(ps0, R, g0, be0, 64)
    Z1, ps1 = _layer_mid(Z0, p0, W1.T)
    p1 = _bn_params(ps1, R, g1, be1, 64)
    M, ps2 = _layer_last(Z1, p1, W2.T)
    p2 = _bn_params(ps2, R, g2, be2, 128)
    new_points = _finalize(M, p2, B, S)

    return (jnp.transpose(new_xyz, (0, 2, 1)), new_points)


# speedup vs baseline: 14.4689x; 9.3525x over previous
import functools

import jax
import jax.numpy as jnp
from jax import lax
from jax.experimental import pallas as pl
from jax.experimental.pallas import tpu as pltpu
from jax.experimental.pallas import tpu_sc as plsc

_NPOINT = 1024
_RADIUS = 0.4
_NSAMPLE = 32
_EPS = 1e-5
_R2 = _RADIUS * _RADIUS

_NC = 2
_NS = 16
_NW = _NC * _NS


def _fps_body(xyz_ref, idx_ref, nxx_ref, nxy_ref, nxz_ref, s1_ref):
    B = xyz_ref.shape[0]
    N = xyz_ref.shape[2]
    S = _NPOINT
    x = xyz_ref[:, 0, :]
    y = xyz_ref[:, 1, :]
    z = xyz_ref[:, 2, :]
    iota = jax.lax.broadcasted_iota(jnp.int32, (B, N), 1)
    iota_s = jax.lax.broadcasted_iota(jnp.int32, (B, S), 1)

    def step(i, carry):
        dist, far, cen, ax, ay, az = carry
        slot = iota_s == i
        cen = jnp.where(slot, far, cen)
        onehot = iota == far
        cx = jnp.sum(jnp.where(onehot, x, 0.0), axis=1, keepdims=True)
        cy = jnp.sum(jnp.where(onehot, y, 0.0), axis=1, keepdims=True)
        cz = jnp.sum(jnp.where(onehot, z, 0.0), axis=1, keepdims=True)
        ax = jnp.where(slot, cx, ax)
        ay = jnp.where(slot, cy, ay)
        az = jnp.where(slot, cz, az)
        dx = x - cx
        dy = y - cy
        dz = z - cz
        d = (dx * dx + dz * dz) + dy * dy
        dist = jnp.minimum(dist, d)
        m = jnp.max(dist, axis=1, keepdims=True)
        far = jnp.min(jnp.where(dist == m, iota, N), axis=1, keepdims=True)
        return dist, far, cen, ax, ay, az

    dist0 = jnp.full((B, N), 1e10, jnp.float32) + x * 0.0
    far0 = (x[:, :1] * 0.0).astype(jnp.int32)
    zs = x[:, :S] * 0.0
    cen0 = zs.astype(jnp.int32)
    _, _, cen, ax, ay, az = jax.lax.fori_loop(
        0, S, step, (dist0, far0, cen0, zs, zs, zs))
    idx_ref[...] = cen
    nxx_ref[...] = ax
    nxy_ref[...] = ay
    nxz_ref[...] = az
    s1_ref[...] = ax * ax + ay * ay + az * az


def _fps_pallas(xyz):
    B, _, N = xyz.shape
    S = _NPOINT
    return pl.pallas_call(
        _fps_body,
        grid=(1,),
        in_specs=[pl.BlockSpec((B, 3, N), lambda i: (0, 0, 0))],
        out_specs=[pl.BlockSpec((B, S), lambda i: (0, 0))] * 5,
        out_shape=[jax.ShapeDtypeStruct((B, S), jnp.int32)]
        + [jax.ShapeDtypeStruct((B, S), jnp.float32)] * 4,
    )(xyz)


_TS_D = 256


def _sqsel_body(cp_ref, xyz_ref, idx_ref):
    N = xyz_ref.shape[2]
    TS = _TS_D
    xyzb = xyz_ref[0]
    x = xyzb[0:1]
    y = xyzb[1:2]
    z = xyzb[2:3]
    s2 = x * x + y * y + z * z
    cpf = cp_ref[0]
    s1col = cpf[:, 7:8]
    cb = cpf.astype(jnp.bfloat16)
    pb = jnp.concatenate(
        [xyzb, jnp.zeros((5, N), jnp.float32)], axis=0).astype(jnp.bfloat16)
    dotp = jnp.dot(cb, pb, preferred_element_type=jnp.float32)
    d = (s1col + s2) - 2.0 * dotp
    iota_n = jax.lax.broadcasted_iota(jnp.int32, (TS, N), 1)
    key = jnp.where(d <= _R2, iota_n, N)
    cols = []
    last = key[:, :1] * 0 - 1
    for _ in range(_NSAMPLE):
        t = jnp.where(key > last, key, N)
        nk = jnp.min(t, axis=1, keepdims=True)
        cols.append(nk)
        last = nk
    idx = jnp.concatenate(cols, axis=1)
    first = idx[:, :1]
    idx_ref[0] = jnp.where(idx == N, first, idx)


def _sqsel(cpack, xyz):
    B, S, _ = cpack.shape
    N = xyz.shape[2]
    return pl.pallas_call(
        _sqsel_body,
        grid=(B, S // _TS_D),
        in_specs=[
            pl.BlockSpec((1, _TS_D, 8), lambda b, j: (b, j, 0)),
            pl.BlockSpec((1, 3, N), lambda b, j: (b, 0, 0)),
        ],
        out_specs=pl.BlockSpec((1, _TS_D, _NSAMPLE), lambda b, j: (b, j, 0)),
        out_shape=jax.ShapeDtypeStruct((B, S, _NSAMPLE), jnp.int32),
    )(cpack, xyz)


_TR = 2048


_SC_CHUNK = 1024


def _sc_gather(idx_flat, T64, T16):
    RK = idx_flat.shape[0]
    per_w = RK // _NW
    sk_per_batch = RK // 8
    n_rows = T64.shape[0] // 8
    n_chunks = per_w // _SC_CHUNK
    mesh = plsc.VectorSubcoreMesh(
        core_axis_name="c", subcore_axis_name="s",
        num_cores=_NC, num_subcores=_NS)

    @functools.partial(
        pl.kernel,
        mesh=mesh,
        out_type=[
            jax.ShapeDtypeStruct((RK, 64), jnp.float32),
            jax.ShapeDtypeStruct((RK, 16), jnp.float32),
        ],
        scratch_types=[
            pltpu.VMEM((per_w,), jnp.int32),
            pltpu.VMEM((_SC_CHUNK, 64), jnp.float32),
            pltpu.VMEM((_SC_CHUNK, 16), jnp.float32),
            pltpu.SemaphoreType.DMA,
            pltpu.SemaphoreType.DMA,
        ],
        compiler_params=pltpu.CompilerParams(use_tc_tiling_on_sc=False),
    )
    def body(idx_hbm, t64_hbm, t16_hbm, outf_hbm, outx_hbm,
             idxv, rows64, rows16, sem1, sem2):
        wid = lax.axis_index("s") * _NC + lax.axis_index("c")
        base = wid * per_w
        boff = (base // sk_per_batch) * n_rows
        pltpu.sync_copy(idx_hbm.at[pl.ds(base, per_w)], idxv)

        def add_body(i, carry):
            idxv[pl.ds(i * 16, 16)] = idxv[pl.ds(i * 16, 16)] + boff
            return carry

        lax.fori_loop(0, per_w // 16, add_body, 0)

        def chunk_body(c, carry):
            ids = idxv.at[pl.ds(c * _SC_CHUNK, _SC_CHUNK)]
            cp1 = pltpu.async_copy(t64_hbm.at[ids], rows64, sem1)
            cp2 = pltpu.async_copy(t16_hbm.at[ids], rows16, sem2)
            cp1.wait()
            cp2.wait()
            dst = pl.ds(base + c * _SC_CHUNK, _SC_CHUNK)
            pltpu.sync_copy(rows64, outf_hbm.at[dst])
            pltpu.sync_copy(rows16, outx_hbm.at[dst])
            return carry

        lax.fori_loop(0, n_chunks, chunk_body, 0)

    return body(idx_flat, T64, T16)


def _layer0_body(f_ref, xg_ref, nx_ref, w_ref, z_ref, ps_ref):
    g = _TR // _NSAMPLE
    dxyz = (xg_ref[...].reshape(g, _NSAMPLE, 16)
            - nx_ref[...][:, None, :]).reshape(_TR, 16)
    x67 = jnp.concatenate([dxyz[:, :3], f_ref[...]], axis=1)
    z = jnp.dot(x67.astype(jnp.bfloat16), w_ref[...].astype(jnp.bfloat16),
                preferred_element_type=jnp.float32)
    z_ref[...] = z
    ps_ref[...] = jnp.stack([jnp.sum(z, axis=0), jnp.sum(z * z, axis=0)])[None]


def _layer0(Fg, Xg, nxyz16, W67):
    R = Fg.shape[0]
    grid = (R // _TR,)
    g = _TR // _NSAMPLE
    return pl.pallas_call(
        _layer0_body,
        grid=grid,
        in_specs=[
            pl.BlockSpec((_TR, 64), lambda i: (i, 0)),
            pl.BlockSpec((_TR, 16), lambda i: (i, 0)),
            pl.BlockSpec((g, 16), lambda i: (i, 0)),
            pl.BlockSpec((67, 64), lambda i: (0, 0)),
        ],
        out_specs=[
            pl.BlockSpec((_TR, 64), lambda i: (i, 0)),
            pl.BlockSpec((1, 2, 64), lambda i: (i, 0, 0)),
        ],
        out_shape=[
            jax.ShapeDtypeStruct((R, 64), jnp.float32),
            jax.ShapeDtypeStruct((R // _TR, 2, 64), jnp.float32),
        ],
    )(Fg, Xg, nxyz16, W67)


def _layer_mid_body(z_ref, p_ref, w_ref, z1_ref, ps_ref):
    x = jax.nn.relu(z_ref[...] * p_ref[0][None, :] + p_ref[1][None, :])
    z1 = jnp.dot(x.astype(jnp.bfloat16), w_ref[...].astype(jnp.bfloat16),
                 preferred_element_type=jnp.float32)
    z1_ref[...] = z1
    ps_ref[...] = jnp.stack([jnp.sum(z1, axis=0), jnp.sum(z1 * z1, axis=0)])[None]


def _layer_mid(Z0, params0, W1_t):
    R = Z0.shape[0]
    grid = (R // _TR,)
    return pl.pallas_call(
        _layer_mid_body,
        grid=grid,
        in_specs=[
            pl.BlockSpec((_TR, 64), lambda i: (i, 0)),
            pl.BlockSpec((8, 64), lambda i: (0, 0)),
            pl.BlockSpec((64, 64), lambda i: (0, 0)),
        ],
        out_specs=[
            pl.BlockSpec((_TR, 64), lambda i: (i, 0)),
            pl.BlockSpec((1, 2, 64), lambda i: (i, 0, 0)),
        ],
        out_shape=[
            jax.ShapeDtypeStruct((R, 64), jnp.float32),
            jax.ShapeDtypeStruct((R // _TR, 2, 64), jnp.float32),
        ],
    )(Z0, params0, W1_t)


def _layer_last_body(z_ref, p_ref, w_ref, m_ref, ps_ref):
    x = jax.nn.relu(z_ref[...] * p_ref[0][None, :] + p_ref[1][None, :])
    z2 = jnp.dot(x.astype(jnp.bfloat16), w_ref[...].astype(jnp.bfloat16),
                 preferred_element_type=jnp.float32)
    ps_ref[...] = jnp.stack([jnp.sum(z2, axis=0), jnp.sum(z2 * z2, axis=0)])[None]
    m_ref[...] = jnp.max(z2.reshape(_TR // _NSAMPLE, _NSAMPLE, 128), axis=1)


def _layer_last(Z1, params1, W2_t):
    R = Z1.shape[0]
    grid = (R // _TR,)
    G = _TR // _NSAMPLE
    return pl.pallas_call(
        _layer_last_body,
        grid=grid,
        in_specs=[
            pl.BlockSpec((_TR, 64), lambda i: (i, 0)),
            pl.BlockSpec((8, 64), lambda i: (0, 0)),
            pl.BlockSpec((64, 128), lambda i: (0, 0)),
        ],
        out_specs=[
            pl.BlockSpec((G, 128), lambda i: (i, 0)),
            pl.BlockSpec((1, 2, 128), lambda i: (i, 0, 0)),
        ],
        out_shape=[
            jax.ShapeDtypeStruct((R // _NSAMPLE, 128), jnp.float32),
            jax.ShapeDtypeStruct((R // _TR, 2, 128), jnp.float32),
        ],
    )(Z1, params1, W2_t)


_TS_OUT = 256


def _finalize_body(m_ref, p_ref, o_ref):
    y = jax.nn.relu(m_ref[...] * p_ref[0][None, :] + p_ref[1][None, :])
    o_ref[...] = y.T[None]


def _finalize(M, params2, B, S):
    grid = (B, S // _TS_OUT)
    return pl.pallas_call(
        _finalize_body,
        grid=grid,
        in_specs=[
            pl.BlockSpec((_TS_OUT, 128), lambda b, j: (b * (S // _TS_OUT) + j, 0)),
            pl.BlockSpec((8, 128), lambda b, j: (0, 0)),
        ],
        out_specs=pl.BlockSpec((1, 128, _TS_OUT), lambda b, j: (b, 0, j)),
        out_shape=jax.ShapeDtypeStruct((B, 128, S), jnp.float32),
    )(M, params2)


def _bn_params(psum, count, g, be, cout):
    s = jnp.sum(psum, axis=0)
    mean = s[0] / count
    var = s[1] / count - mean * mean
    inv = g * jax.lax.rsqrt(var + _EPS)
    a = inv
    c = be - mean * inv
    pad = jnp.zeros((6, cout), jnp.float32)
    return jnp.concatenate([a[None], c[None], pad], axis=0)


def kernel(xyz, points, W0, b0, g0, be0, W1, b1, g1, be1, W2, b2, g2, be2):
    B, _, N = xyz.shape
    C = points.shape[1]
    S, K = _NPOINT, _NSAMPLE
    R = B * S * K

    fps_idx, nxx, nxy, nxz, s1 = _fps_pallas(xyz)
    new_xyz_t = jnp.stack([nxx, nxy, nxz], axis=1)

    zero = jnp.zeros_like(s1)
    cpack = jnp.stack(
        [nxx, nxy, nxz, zero, zero, zero, zero, s1],
        axis=-1)
    idx = _sqsel(cpack, xyz)

    pts_rows = jnp.transpose(points, (0, 2, 1)).reshape(B * N, C)
    xyz_rows = jnp.transpose(xyz, (0, 2, 1)).reshape(B * N, 3)
    xyz16 = jnp.concatenate(
        [xyz_rows, jnp.zeros((B * N, 13), jnp.float32)], axis=1)

    Fg, Xg = _sc_gather(idx.reshape(R), pts_rows, xyz16)

    nxyz16 = jnp.stack(
        [nxx, nxy, nxz] + [zero] * 13, axis=-1).reshape(B * S, 16)
    Z0, ps0 = _layer0(Fg, Xg, nxyz16, W0.T)
    p0 = _bn_params(ps0, R, g0, be0, 64)
    Z1, ps1 = _layer_mid(Z0, p0, W1.T)
    p1 = _bn_params(ps1, R, g1, be1, 64)
    M, ps2 = _layer_last(Z1, p1, W2.T)
    p2 = _bn_params(ps2, R, g2, be2, 128)
    new_points = _finalize(M, p2, B, S)

    return (new_xyz_t, new_points)
